# baseline (speedup 1.0000x reference)
import functools
import jax
import jax.numpy as jnp
from jax import lax
from jax.experimental import pallas as pl
from jax.experimental.pallas import tpu as pltpu
from jax.experimental.pallas import tpu_sc as plsc


def kernel(x, emb):
    b, s, d = x.shape
    NC, NS = 2, 16
    NW = NC * NS
    rows_per_w = s // NW

    CH = 88
    sizes = []
    rem = rows_per_w
    while rem > 0:
        c = min(rem, CH)
        sizes.append(c)
        rem -= c
    offs = [sum(sizes[:i]) for i in range(len(sizes))]
    n_chunks = len(sizes)

    mesh = plsc.VectorSubcoreMesh(core_axis_name="c", subcore_axis_name="s")

    @functools.partial(
        pl.kernel,
        mesh=mesh,
        out_type=jax.ShapeDtypeStruct((b, s, d), jnp.float32),
        scratch_types=[
            pltpu.VMEM((CH, d), jnp.float32),
            pltpu.SemaphoreType.DMA,
            pltpu.SemaphoreType.DMA,
        ],
    )
    def sc_copy(emb_hbm, out_hbm, buf, rsem, wsem):
        wid = lax.axis_index("s") * NC + lax.axis_index("c")
        base = wid * rows_per_w

        for i in range(n_chunks):
            r = pltpu.make_async_copy(
                emb_hbm.at[pl.ds(base + offs[i], sizes[i])],
                buf.at[pl.ds(0, sizes[i])], rsem)
            r.start()
            r.wait()
            ws = [
                pltpu.make_async_copy(
                    buf.at[pl.ds(0, sizes[i])],
                    out_hbm.at[bi].at[pl.ds(base + offs[i], sizes[i])],
                    wsem)
                for bi in range(b)
            ]
            for w in ws:
                w.start()
            for w in ws:
                w.wait()

    return sc_copy(emb)


# speedup vs baseline: 1.0111x; 1.0111x over previous
import functools
import jax
import jax.numpy as jnp
from jax import lax
from jax.experimental import pallas as pl
from jax.experimental.pallas import tpu as pltpu
from jax.experimental.pallas import tpu_sc as plsc


def kernel(x, emb):
    b, s, d = x.shape
    NC, NS = 2, 16
    NW = NC * NS
    rows_per_w = s // NW

    CH = 120
    sizes = []
    rem = rows_per_w
    while rem > 0:
        c = min(rem, CH)
        sizes.append(c)
        rem -= c
    sizes = sizes[::-1]
    offs = [sum(sizes[:i]) for i in range(len(sizes))]
    n_chunks = len(sizes)

    mesh = plsc.VectorSubcoreMesh(core_axis_name="c", subcore_axis_name="s")

    @functools.partial(
        pl.kernel,
        mesh=mesh,
        out_type=jax.ShapeDtypeStruct((b, s, d), jnp.float32),
        scratch_types=[
            pltpu.VMEM((CH, d), jnp.float32),
            pltpu.SemaphoreType.DMA,
            pltpu.SemaphoreType.DMA,
        ],
    )
    def sc_copy(emb_hbm, out_hbm, buf, rsem, wsem):
        wid = lax.axis_index("s") * NC + lax.axis_index("c")
        base = wid * rows_per_w

        for i in range(n_chunks):
            r = pltpu.make_async_copy(
                emb_hbm.at[pl.ds(base + offs[i], sizes[i])],
                buf.at[pl.ds(0, sizes[i])], rsem)
            r.start()
            r.wait()
            ws = [
                pltpu.make_async_copy(
                    buf.at[pl.ds(0, sizes[i])],
                    out_hbm.at[bi].at[pl.ds(base + offs[i], sizes[i])],
                    wsem)
                for bi in range(b)
            ]
            for w in ws:
                w.start()
            for w in ws:
                w.wait()

    return sc_copy(emb)
